# baseline (speedup 1.0000x reference)
import functools

import jax
import jax.numpy as jnp
from jax import lax
from jax.experimental import pallas as pl
from jax.experimental.pallas import tpu as pltpu
from jax.experimental.pallas import tpu_sc as plsc

CUT2 = 100.0
K = 32
BIG = 1.0e30
RA = 256
CCA = 512
NW = 32
PIECE = 512


def _fill_body(clo_ref, chi_ref, posp_blk, posTp, bat_blk, bat_row, d2_out, *, N):
    g = pl.program_id(0)
    c_lo = clo_ref[g]
    c_hi = chi_ref[g]

    xi = posp_blk[:, 0:1]
    yi = posp_blk[:, 1:2]
    zi = posp_blk[:, 2:3]
    sqi = (xi * xi + yi * yi) + zi * zi
    bi = bat_blk[:, 0:1]
    rowg = g * RA + lax.broadcasted_iota(jnp.int32, (RA, 1), 0)

    def compute_chunk(c, _):
        cols = pl.ds(c * CCA, CCA)
        p = posTp[:, cols]
        xj = p[0:1, :]
        yj = p[1:2, :]
        zj = p[2:3, :]
        sqj = (xj * xj + yj * yj) + zj * zj
        gram = jnp.dot(posp_blk[...], p, preferred_element_type=jnp.float32)
        d2 = jnp.maximum((sqi + sqj) - 2.0 * gram, 0.0)
        colid = c * CCA + lax.broadcasted_iota(jnp.int32, (RA, CCA), 1)
        same = bi == bat_row[0:1, cols]
        keep = same & (colid != rowg)
        d2_out[:, cols] = jnp.where(keep, d2, BIG)
        return 0

    lax.fori_loop(c_lo, c_hi, compute_chunk, 0)


def _fill(N):
    grid_spec = pltpu.PrefetchScalarGridSpec(
        num_scalar_prefetch=2,
        grid=(N // RA,),
        in_specs=[
            pl.BlockSpec((RA, 8), lambda i, clo, chi: (i, 0)),
            pl.BlockSpec((8, N), lambda i, clo, chi: (0, 0)),
            pl.BlockSpec((RA, 1), lambda i, clo, chi: (i, 0)),
            pl.BlockSpec((1, N), lambda i, clo, chi: (0, 0)),
        ],
        out_specs=[pl.BlockSpec((RA, N), lambda i, clo, chi: (i, 0))],
    )
    return pl.pallas_call(
        functools.partial(_fill_body, N=N),
        grid_spec=grid_spec,
        out_shape=[jax.ShapeDtypeStruct((N, N), jnp.float32)],
    )


def _sc_select(N):
    RPW = N // NW
    NCH = N // 16
    mesh = plsc.VectorSubcoreMesh(core_axis_name="c", subcore_axis_name="s")

    @functools.partial(
        pl.kernel,
        mesh=mesh,
        out_type=[
            jax.ShapeDtypeStruct((N * K,), jnp.int32),
            jax.ShapeDtypeStruct((N * K,), jnp.float32),
        ],
        scratch_types=[
            pltpu.VMEM((N + 16,), jnp.float32),
            pltpu.VMEM((N + 16,), jnp.float32),
            pltpu.VMEM((N + 16,), jnp.float32),
            pltpu.VMEM((N + 16,), jnp.float32),
            pltpu.VMEM((NCH + 16,), jnp.float32),
            pltpu.VMEM((NW + 16,), jnp.int32),
            pltpu.VMEM((NW + 16,), jnp.int32),
            pltpu.VMEM((RPW * K + 16,), jnp.int32),
            pltpu.VMEM((RPW * K + 16,), jnp.float32),
            pltpu.VMEM((32,), jnp.float32),
            pltpu.VMEM((32,), jnp.int32),
        ],
    )
    def body(d2_hbm, px_hbm, py_hbm, pz_hbm, wlo_hbm, whi_hbm, oidx_hbm, odd_hbm,
             px_v, py_v, pz_v, row_v, cm_v, wlo_v, whi_v, oi_v, od_v, vbf, vbi):
        lane = lax.broadcasted_iota(jnp.int32, (16,), 0)
        lane0 = lane == 0

        def sload(ref, idx):
            return ref[pl.ds(idx, 16)][0]

        def sstore(ref, idx, val):
            base = (idx // 16) * 16
            off = idx - base
            v = ref[pl.ds(base, 16)]
            ref[pl.ds(base, 16)] = jnp.where(
                lane == off, jnp.zeros((16,), ref.dtype) + val, v
            )

        vbf[pl.ds(16, 16)] = jnp.full((16,), BIG, jnp.float32)
        vbi[pl.ds(16, 16)] = jnp.full((16,), 1073741824, jnp.int32)

        def vminf(x):
            for sh in (8, 4, 2, 1):
                vbf[pl.ds(0, 16)] = x
                x = jnp.minimum(x, vbf[pl.ds(sh, 16)])
            return x[0]

        def vmini(x):
            for sh in (8, 4, 2, 1):
                vbi[pl.ds(0, 16)] = x
                x = jnp.minimum(x, vbi[pl.ds(sh, 16)])
            return x[0]

        cid = lax.axis_index("c")
        sid = lax.axis_index("s")
        wid = sid * 2 + cid
        pltpu.sync_copy(px_hbm, px_v.at[pl.ds(0, N)])
        pltpu.sync_copy(py_hbm, py_v.at[pl.ds(0, N)])
        pltpu.sync_copy(pz_hbm, pz_v.at[pl.ds(0, N)])
        pltpu.sync_copy(wlo_hbm, wlo_v.at[pl.ds(0, NW)])
        pltpu.sync_copy(whi_hbm, whi_v.at[pl.ds(0, NW)])
        w0 = pl.multiple_of(sload(wlo_v, wid), PIECE)
        w1 = pl.multiple_of(sload(whi_v, wid), PIECE)
        npc = (w1 - w0) // PIECE
        nv = (w1 - w0) // 256
        r0 = wid * RPW

        def row_fn(il, _):
            i = r0 + il

            def dma_fn(t, _):
                off = pl.multiple_of(i * N + w0 + t * PIECE, PIECE)
                pltpu.sync_copy(
                    d2_hbm.at[pl.ds(off, PIECE)],
                    row_v.at[pl.ds(t * PIECE, PIECE)],
                )
                return 0

            lax.fori_loop(0, npc, dma_fn, 0)

            def cmins(t, _):
                sstore(cm_v, t, vminf(row_v[pl.ds(t * 16, 16)]))
                return 0

            lax.fori_loop(0, nv * 16, cmins, 0)
            xi = sload(px_v, i)
            yi = sload(py_v, i)
            zi = sload(pz_v, i)

            def sel_fn(k, _):
                def scan_fn(u, carry):
                    bv, bix = carry
                    c = cm_v[pl.ds(u * 16, 16)]
                    cidx = u * 16 + lane
                    t = c < bv
                    return jnp.where(t, c, bv), jnp.where(t, cidx, bix)

                bv0 = jnp.full((16,), BIG, jnp.float32)
                bi0 = jnp.full((16,), 100000, jnp.int32)
                bv, bix = lax.fori_loop(0, nv, scan_fn, (bv0, bi0))
                mv = vminf(bv)
                ci = vmini(jnp.where(bv == (jnp.zeros((16,), jnp.float32) + mv), bix, 100000))
                d = row_v[pl.ds(ci * 16, 16)]
                li = vmini(jnp.where(d == (jnp.zeros((16,), jnp.float32) + mv), lane, 15))
                p = ci * 16 + li
                jglob = w0 + p
                ok = mv <= CUT2
                xj = sload(px_v, jglob)
                yj = sload(py_v, jglob)
                zj = sload(pz_v, jglob)
                dx = xi - xj
                dy = yi - yj
                dz = zi - zj
                dd = (dx * dx + dy * dy) + dz * dz
                sstore(oi_v, il * K + k, jnp.where(ok, jglob, i))
                sstore(od_v, il * K + k, jnp.where(ok, dd, -1.0))
                sstore(row_v, p, BIG)
                sstore(cm_v, ci, vminf(row_v[pl.ds(ci * 16, 16)]))
                return 0

            lax.fori_loop(0, K, sel_fn, 0)
            return 0

        lax.fori_loop(0, RPW, row_fn, 0)
        pltpu.sync_copy(oi_v.at[pl.ds(0, RPW * K)], oidx_hbm.at[pl.ds(r0 * K, RPW * K)])
        pltpu.sync_copy(od_v.at[pl.ds(0, RPW * K)], odd_hbm.at[pl.ds(r0 * K, RPW * K)])

    return body


def _sqrt_body(dd_ref, w_ref):
    dd = dd_ref[...]
    w_ref[...] = jnp.where(dd > 0.0, jnp.sqrt(jnp.where(dd > 0.0, dd, 1.0)), 0.0)


def _sqrt_ep(N):
    return pl.pallas_call(
        _sqrt_body,
        out_shape=jax.ShapeDtypeStruct((N, K), jnp.float32),
    )


def kernel(pos, batch):
    N = pos.shape[0]
    nblk = N // RA
    batch32 = batch.astype(jnp.int32)
    posp = jnp.pad(pos, ((0, 0), (0, 5)))
    posTp = posp.T
    bat_col = batch32.reshape(N, 1)
    bat_row = batch32.reshape(1, N)
    r0 = jnp.arange(nblk, dtype=jnp.int32) * RA
    b_first = batch32[r0]
    b_last = batch32[r0 + RA - 1]
    lo = jnp.searchsorted(batch32, b_first, side="left").astype(jnp.int32)
    hi = jnp.searchsorted(batch32, b_last, side="right").astype(jnp.int32)
    clo = lo // CCA
    chi = (hi + CCA - 1) // CCA
    (d2win,) = _fill(N)(clo, chi, posp, posTp, bat_col, bat_row)
    px = pos[:, 0] + 0.0
    py = pos[:, 1] + 0.0
    pz = pos[:, 2] + 0.0
    oidx, odd = _sc_select(N)(d2win.reshape(-1), px, py, pz, clo * CCA, chi * CCA)
    w = _sqrt_ep(N)(odd.reshape(N, K))
    centers = jnp.broadcast_to(jnp.arange(N, dtype=jnp.int32)[:, None], (N, K))
    edge_index = jnp.stack([oidx, centers.reshape(-1)]).astype(jnp.int64)
    edge_weight = w.reshape(-1)
    return edge_index, edge_weight


# speedup vs baseline: 2.2900x; 2.2900x over previous
import functools

import jax
import jax.numpy as jnp
from jax.experimental import pallas as pl
from jax.experimental.pallas import tpu as pltpu

CUT2 = 100.0
K = 32
BIG = 1.0e30


def _tc_body(clo_ref, chi_ref, posp_blk, posTp, bat_blk, bat_row, idx_out, w_out,
             d2_ref, dd_ref, *, R, N, CC):
    g = pl.program_id(0)
    c_lo = clo_ref[g]
    c_hi = chi_ref[g]

    xi = posp_blk[:, 0:1]
    yi = posp_blk[:, 1:2]
    zi = posp_blk[:, 2:3]
    sqi = (xi * xi + yi * yi) + zi * zi
    bi = bat_blk[:, 0:1]
    rowg = g * R + jax.lax.broadcasted_iota(jnp.int32, (R, 1), 0)

    def compute_chunk(c, _):
        cols = pl.ds(c * CC, CC)
        p = posTp[:, cols]
        xj = p[0:1, :]
        yj = p[1:2, :]
        zj = p[2:3, :]
        sqj = (xj * xj + yj * yj) + zj * zj
        gram = jnp.dot(posp_blk[...], p, preferred_element_type=jnp.float32)
        d2 = jnp.maximum((sqi + sqj) - 2.0 * gram, 0.0)
        dx = xi - xj
        dy = yi - yj
        dz = zi - zj
        dd = (dx * dx + dy * dy) + dz * dz
        colid = c * CC + jax.lax.broadcasted_iota(jnp.int32, (R, CC), 1)
        same = bi == bat_row[0:1, cols]
        keep = same & (colid != rowg)
        d2_ref[:, cols] = jnp.where(keep, d2, BIG)
        dd_ref[:, cols] = dd
        return 0

    jax.lax.fori_loop(c_lo, c_hi, compute_chunk, 0)

    slot = jax.lax.broadcasted_iota(jnp.int32, (R, K), 1)

    def select_one(k, state):
        am_prev, idx_acc, w_acc = state

        def scan_chunk(c, carry):
            mv, am, dv = carry
            cols = pl.ds(c * CC, CC)
            dc = d2_ref[:, cols]
            colid = c * CC + jax.lax.broadcasted_iota(jnp.int32, (R, CC), 1)
            dc = jnp.where(colid == am_prev, BIG, dc)
            d2_ref[:, cols] = dc
            cmin = jnp.min(dc, axis=1, keepdims=True)
            carg = jnp.min(jnp.where(dc == cmin, colid, N), axis=1, keepdims=True)
            cdd = jnp.min(
                jnp.where(colid == carg, dd_ref[:, cols], BIG), axis=1, keepdims=True
            )
            better = cmin < mv
            return (
                jnp.where(better, cmin, mv),
                jnp.where(better, carg, am),
                jnp.where(better, cdd, dv),
            )

        mv0 = jnp.full((R, 1), BIG, jnp.float32)
        am0 = jnp.full((R, 1), -2, jnp.int32)
        mv, am, dv = jax.lax.fori_loop(c_lo, c_hi, scan_chunk, (mv0, am0, mv0))

        valid = mv <= CUT2
        iv = jnp.where(valid, am, rowg)
        wv = jnp.where(
            valid & (dv > 0.0), jnp.sqrt(jnp.where(dv > 0.0, dv, 1.0)), 0.0
        )
        here = slot == k
        idx_acc = jnp.where(here, iv, idx_acc)
        w_acc = jnp.where(here, wv, w_acc)
        return am, idx_acc, w_acc

    _, idx_acc, w_acc = jax.lax.fori_loop(
        0,
        K,
        select_one,
        (
            jnp.full((R, 1), -2, jnp.int32),
            jnp.zeros((R, K), jnp.int32),
            jnp.zeros((R, K), jnp.float32),
        ),
    )
    idx_out[...] = idx_acc
    w_out[...] = w_acc


def _build(N, R, CC):
    nblk = N // R
    grid_spec = pltpu.PrefetchScalarGridSpec(
        num_scalar_prefetch=2,
        grid=(nblk,),
        in_specs=[
            pl.BlockSpec((R, 8), lambda i, clo, chi: (i, 0)),
            pl.BlockSpec((8, N), lambda i, clo, chi: (0, 0)),
            pl.BlockSpec((R, 1), lambda i, clo, chi: (i, 0)),
            pl.BlockSpec((1, N), lambda i, clo, chi: (0, 0)),
        ],
        out_specs=[
            pl.BlockSpec((R, K), lambda i, clo, chi: (i, 0)),
            pl.BlockSpec((R, K), lambda i, clo, chi: (i, 0)),
        ],
        scratch_shapes=[
            pltpu.VMEM((R, N), jnp.float32),
            pltpu.VMEM((R, N), jnp.float32),
        ],
    )
    return pl.pallas_call(
        functools.partial(_tc_body, R=R, N=N, CC=CC),
        grid_spec=grid_spec,
        out_shape=[
            jax.ShapeDtypeStruct((N, K), jnp.int32),
            jax.ShapeDtypeStruct((N, K), jnp.float32),
        ],
    )


def kernel(pos, batch):
    N = pos.shape[0]
    R = 256 if N % 256 == 0 else 128
    CC = 1024
    nblk = N // R
    batch32 = batch.astype(jnp.int32)
    posp = jnp.pad(pos, ((0, 0), (0, 5)))
    posTp = posp.T
    bat_col = batch32.reshape(N, 1)
    bat_row = batch32.reshape(1, N)
    r0 = jnp.arange(nblk, dtype=jnp.int32) * R
    b_first = batch32[r0]
    b_last = batch32[r0 + R - 1]
    lo = jnp.searchsorted(batch32, b_first, side="left").astype(jnp.int32)
    hi = jnp.searchsorted(batch32, b_last, side="right").astype(jnp.int32)
    clo = lo // CC
    chi = (hi + CC - 1) // CC
    idx, w = _build(N, R, CC)(clo, chi, posp, posTp, bat_col, bat_row)
    centers = jnp.broadcast_to(jnp.arange(N, dtype=jnp.int32)[:, None], (N, K))
    row = idx.reshape(-1)
    col = centers.reshape(-1)
    edge_index = jnp.stack([row, col]).astype(jnp.int64)
    edge_weight = w.reshape(-1)
    return edge_index, edge_weight
